# Initial kernel scaffold; baseline (speedup 1.0000x reference)
import functools

import jax
import jax.numpy as jnp
from jax import lax
from jax.experimental import pallas as pl
from jax.experimental.pallas import tpu as pltpu
from jax.experimental.pallas import tpu_sc as plsc

N = 10000
E = 320000
D = 128
H = 64
Bn = 1000
NBLK = N // Bn

CH = 128
NCH = E // CH
NS = 16
CPS = NCH // NS
EXTRA = NCH - CPS * NS
NB = 3
RPS = (2 * N) // NS
ZCH = 125


def _leaky(v):
    return jnp.where(v >= 0, v, 0.01 * v)


def _table_write(x, wr_ref, t_ref):
    xr0 = jnp.dot(x, wr_ref[0])
    xr1 = jnp.dot(x, wr_ref[1])
    t_ref[0] = xr0[:, :H]
    t_ref[1] = xr1[:, :H]
    t_ref[2] = xr0[:, H:]
    t_ref[3] = xr1[:, H:]


def _k1_body(cat_ref, wc_ref, bc_ref, wi_ref, bi_ref, wr_ref, x_ref, t_ref):
    c = _leaky(jnp.dot(cat_ref[...], wc_ref[...]) + bc_ref[...])
    x = _leaky(jnp.dot(c, wi_ref[...]) + bi_ref[...])
    x_ref[...] = x
    _table_write(x, wr_ref, t_ref)


def _combine(x, wroot_ref, br_ref, a00, a01, a10, a11, c0, c1):
    cnt0 = jnp.maximum(c0[0][:, 0:1], 1.0)
    cnt1 = jnp.maximum(c1[0][:, 0:1], 1.0)
    lo = a00[0] / cnt0 + a01[0] / cnt1
    hi = a10[0] / cnt0 + a11[0] / cnt1
    return (jnp.dot(x, wroot_ref[...]) + br_ref[...]
            + jnp.concatenate([lo, hi], axis=1))


def _k2_body(x_ref, wroot_ref, br_ref, a00, a01, a10, a11, c0, c1, wr_ref,
             x2_ref, t_ref):
    x2 = _combine(x_ref[...], wroot_ref, br_ref, a00, a01, a10, a11, c0, c1)
    x2_ref[...] = x2
    _table_write(x2, wr_ref, t_ref)


def _k3_body(x_ref, wroot_ref, br_ref, a00, a01, a10, a11, c0, c1,
             wo1_ref, bo1_ref, wo2_ref, bo2_ref, out_ref):
    x3 = _combine(x_ref[...], wroot_ref, br_ref, a00, a01, a10, a11, c0, c1)
    x4 = _leaky(jnp.dot(x3, wo1_ref[...]) + bo1_ref[...])
    out_ref[...] = jnp.dot(x4, wo2_ref[...]) + bo2_ref[...]


def _full(shape):
    return pl.BlockSpec(shape, lambda i: (0,) * len(shape))


def _agg_specs():
    return [
        pl.BlockSpec((1, Bn, H), lambda i: (0, i, 0)),
        pl.BlockSpec((1, Bn, H), lambda i: (0, NBLK + i, 0)),
        pl.BlockSpec((1, Bn, H), lambda i: (1, i, 0)),
        pl.BlockSpec((1, Bn, H), lambda i: (1, NBLK + i, 0)),
    ]


def _cnt_specs():
    return [
        pl.BlockSpec((1, Bn, 16), lambda i: (0, i, 0)),
        pl.BlockSpec((1, Bn, 16), lambda i: (0, NBLK + i, 0)),
    ]


_k1 = pl.pallas_call(
    _k1_body,
    grid=(NBLK,),
    in_specs=[
        pl.BlockSpec((Bn, 11), lambda i: (i, 0)),
        _full((11, D)), _full((1, D)), _full((D, D)), _full((1, D)),
        _full((2, D, D)),
    ],
    out_specs=[
        pl.BlockSpec((Bn, D), lambda i: (i, 0)),
        pl.BlockSpec((4, Bn, H), lambda i: (0, i, 0)),
    ],
    out_shape=[
        jax.ShapeDtypeStruct((N, D), jnp.float32),
        jax.ShapeDtypeStruct((4, N, H), jnp.float32),
    ],
)

_k2 = pl.pallas_call(
    _k2_body,
    grid=(NBLK,),
    in_specs=[
        pl.BlockSpec((Bn, D), lambda i: (i, 0)),
        _full((D, D)), _full((1, D)),
        *_agg_specs(), *_cnt_specs(),
        _full((2, D, D)),
    ],
    out_specs=[
        pl.BlockSpec((Bn, D), lambda i: (i, 0)),
        pl.BlockSpec((4, Bn, H), lambda i: (0, i, 0)),
    ],
    out_shape=[
        jax.ShapeDtypeStruct((N, D), jnp.float32),
        jax.ShapeDtypeStruct((4, N, H), jnp.float32),
    ],
)

_k3 = pl.pallas_call(
    _k3_body,
    grid=(NBLK,),
    in_specs=[
        pl.BlockSpec((Bn, D), lambda i: (i, 0)),
        _full((D, D)), _full((1, D)),
        *_agg_specs(), *_cnt_specs(),
        _full((D, D)), _full((1, D)), _full((D, 2)), _full((1, 2)),
    ],
    out_specs=pl.BlockSpec((Bn, 2), lambda i: (i, 0)),
    out_shape=jax.ShapeDtypeStruct((N, 2), jnp.float32),
)


def _sc_body(with_cnt, tab, gx, sx, *rest):
    if with_cnt:
        (agg_out, cnt_out, acc, cntacc, ones, gall, sall, rows,
         gex, sex, sem0, sem1, sem2) = rest
    else:
        (agg_out, acc, gall, sall, rows, gex, sex, sem0, sem1, sem2) = rest
        cnt_out = cntacc = ones = None
    sems = [sem0, sem1, sem2]
    c = lax.axis_index("c")
    s = lax.axis_index("s")

    def _zr(i, carry):
        def _zc(j, carry2):
            rows[0, i, pl.ds(j * 16, 16)] = jnp.zeros((16,), jnp.float32)
            return carry2
        return lax.fori_loop(0, H // 16, _zc, carry)
    lax.fori_loop(0, CH, _zr, 0)

    if with_cnt:
        def _zo(i, carry):
            ones[i, pl.ds(0, 16)] = jnp.zeros((16,), jnp.float32)
            return carry
        lax.fori_loop(0, CH, _zo, 0)

    def _za(z, carry):
        off = s * RPS + z * ZCH
        pltpu.sync_copy(rows.at[0, pl.ds(0, ZCH)], acc.at[pl.ds(off, ZCH)])
        if with_cnt:
            pltpu.sync_copy(ones.at[pl.ds(0, ZCH)], cntacc.at[pl.ds(off, ZCH)])
        return carry
    lax.fori_loop(0, RPS // ZCH, _za, 0)

    if with_cnt:
        def _so(i, carry):
            ones[i, pl.ds(0, 16)] = jnp.ones((16,), jnp.float32)
            return carry
        lax.fori_loop(0, CH, _so, 0)

    pltpu.sync_copy(gx.at[c, pl.ds(s * CPS, CPS)], gall)
    pltpu.sync_copy(sx.at[pl.ds(s * CPS, CPS)], sall)

    @pl.when(s < EXTRA)
    def _():
        pltpu.sync_copy(gx.at[c, pl.ds(NS * CPS + s, 1)], gex)
        pltpu.sync_copy(sx.at[pl.ds(NS * CPS + s, 1)], sex)

    plsc.subcore_barrier()

    def _outer(i0, carry):
        base = i0 * NB
        descs = []
        for b in range(NB):
            descs.append(
                pltpu.async_copy(tab.at[gall.at[base + b]], rows.at[b],
                                 sems[b]))
        for b in range(NB):
            descs[b].wait()
            pltpu.sync_copy(rows.at[b], acc.at[sall.at[base + b]], add=True)
            if with_cnt:
                pltpu.sync_copy(ones, cntacc.at[sall.at[base + b]], add=True)
        return carry
    lax.fori_loop(0, CPS // NB, _outer, 0)

    @pl.when(s < EXTRA)
    def _():
        d = pltpu.async_copy(tab.at[gex.at[0]], rows.at[0], sems[0])
        d.wait()
        pltpu.sync_copy(rows.at[0], acc.at[sex.at[0]], add=True)
        if with_cnt:
            pltpu.sync_copy(ones, cntacc.at[sex.at[0]], add=True)

    plsc.subcore_barrier()

    pltpu.sync_copy(acc.at[pl.ds(s * RPS, RPS)],
                    agg_out.at[c, pl.ds(s * RPS, RPS)])
    if with_cnt:
        pltpu.sync_copy(cntacc.at[pl.ds(s * RPS, RPS)],
                        cnt_out.at[c, pl.ds(s * RPS, RPS)])


def _make_sc(with_cnt):
    out_type = [jax.ShapeDtypeStruct((2, 2 * N, H), jnp.float32)]
    scratch = [
        pltpu.VMEM_SHARED((2 * N, H), jnp.float32),
    ]
    if with_cnt:
        out_type.append(jax.ShapeDtypeStruct((2, 2 * N, 16), jnp.float32))
        scratch += [
            pltpu.VMEM_SHARED((2 * N, 16), jnp.float32),
            pltpu.VMEM((CH, 16), jnp.float32),
        ]
    scratch += [
        pltpu.VMEM((CPS, CH), jnp.int32),
        pltpu.VMEM((CPS, CH), jnp.int32),
        pltpu.VMEM((NB, CH, H), jnp.float32),
        pltpu.VMEM((1, CH), jnp.int32),
        pltpu.VMEM((1, CH), jnp.int32),
        pltpu.SemaphoreType.DMA,
        pltpu.SemaphoreType.DMA,
        pltpu.SemaphoreType.DMA,
    ]
    return pl.kernel(
        functools.partial(_sc_body, with_cnt),
        out_type=tuple(out_type) if with_cnt else out_type[0],
        mesh=plsc.VectorSubcoreMesh(core_axis_name="c", subcore_axis_name="s"),
        scratch_types=scratch,
    )


_sc1 = _make_sc(True)
_sc2 = _make_sc(False)


def kernel(des, tweet, num_prop, cat_prop, edge_index, edge_type,
           W_cat, b_cat, W_in, b_in, W_rel, W_root, b_rgcn,
           W_o1, b_o1, W_o2, b_o2):
    src = edge_index[0].astype(jnp.int32)
    dst = edge_index[1].astype(jnp.int32)
    et = edge_type.astype(jnp.int32)
    g0 = et * N + src
    gx = jnp.stack([g0, g0 + 2 * N]).reshape(2, NCH, CH)
    sx = (et * N + dst).reshape(NCH, CH)

    bc = b_cat.reshape(1, D)
    bi = b_in.reshape(1, D)
    br = b_rgcn.reshape(1, D)
    bo1 = b_o1.reshape(1, D)
    bo2 = b_o2.reshape(1, 2)

    x1, t1 = _k1(cat_prop, W_cat, bc, W_in, bi, W_rel)
    agg1, cnt16 = _sc1(t1.reshape(4 * N, H), gx, sx)
    x2, t2 = _k2(x1, W_root, br, agg1, agg1, agg1, agg1, cnt16, cnt16, W_rel)
    agg2 = _sc2(t2.reshape(4 * N, H), gx, sx)
    return _k3(x2, W_root, br, agg2, agg2, agg2, agg2, cnt16, cnt16,
               W_o1, bo1, W_o2, bo2)


# speedup vs baseline: 7.8873x; 7.8873x over previous
import functools

import jax
import jax.numpy as jnp
from jax import lax
from jax.experimental import pallas as pl
from jax.experimental.pallas import tpu as pltpu
from jax.experimental.pallas import tpu_sc as plsc

N = 10000
E = 320000
D = 128
H = 64
Bn = 1000
NBLK = N // Bn

CH = 128
NS = 16
NCH = 2560
EPAD = NCH * CH
CPS = NCH // NS
GRP = 8
NGRP = CPS // GRP
NB = 3
ACC_R = 2 * N + 96
NZCH = ACC_R // CH
WCH = 1000
NWCH = (2 * N) // WCH


def _leaky(v):
    return jnp.where(v >= 0, v, 0.01 * v)


def _table_write(x, wr_ref, t_ref):
    xr0 = jnp.dot(x, wr_ref[0])
    xr1 = jnp.dot(x, wr_ref[1])
    t_ref[0] = xr0[:, :H]
    t_ref[1] = xr1[:, :H]
    t_ref[2] = xr0[:, H:]
    t_ref[3] = xr1[:, H:]


def _k1_body(cat_ref, wc_ref, bc_ref, wi_ref, bi_ref, wr_ref, x_ref, t_ref):
    c = _leaky(jnp.dot(cat_ref[...], wc_ref[...]) + bc_ref[...])
    x = _leaky(jnp.dot(c, wi_ref[...]) + bi_ref[...])
    x_ref[...] = x
    _table_write(x, wr_ref, t_ref)


def _combine(x, wroot_ref, br_ref, a00, a01, a10, a11, c0, c1):
    cnt0 = jnp.maximum(c0[0][:, 0:1], 1.0)
    cnt1 = jnp.maximum(c1[0][:, 0:1], 1.0)
    lo = a00[0] / cnt0 + a01[0] / cnt1
    hi = a10[0] / cnt0 + a11[0] / cnt1
    return (jnp.dot(x, wroot_ref[...]) + br_ref[...]
            + jnp.concatenate([lo, hi], axis=1))


def _k2_body(x_ref, wroot_ref, br_ref, a00, a01, a10, a11, c0, c1, wr_ref,
             x2_ref, t_ref):
    x2 = _combine(x_ref[...], wroot_ref, br_ref, a00, a01, a10, a11, c0, c1)
    x2_ref[...] = x2
    _table_write(x2, wr_ref, t_ref)


def _k3_body(x_ref, wroot_ref, br_ref, a00, a01, a10, a11, c0, c1,
             wo1_ref, bo1_ref, wo2_ref, bo2_ref, out_ref):
    x3 = _combine(x_ref[...], wroot_ref, br_ref, a00, a01, a10, a11, c0, c1)
    x4 = _leaky(jnp.dot(x3, wo1_ref[...]) + bo1_ref[...])
    out_ref[...] = jnp.dot(x4, wo2_ref[...]) + bo2_ref[...]


def _full(shape):
    return pl.BlockSpec(shape, lambda i: (0,) * len(shape))


def _agg_specs():
    return [
        pl.BlockSpec((1, Bn, H), lambda i: (0, i, 0)),
        pl.BlockSpec((1, Bn, H), lambda i: (0, NBLK + i, 0)),
        pl.BlockSpec((1, Bn, H), lambda i: (1, i, 0)),
        pl.BlockSpec((1, Bn, H), lambda i: (1, NBLK + i, 0)),
    ]


def _cnt_specs():
    return [
        pl.BlockSpec((1, Bn, 16), lambda i: (0, i, 0)),
        pl.BlockSpec((1, Bn, 16), lambda i: (0, NBLK + i, 0)),
    ]


_k1 = pl.pallas_call(
    _k1_body,
    grid=(NBLK,),
    in_specs=[
        pl.BlockSpec((Bn, 11), lambda i: (i, 0)),
        _full((11, D)), _full((1, D)), _full((D, D)), _full((1, D)),
        _full((2, D, D)),
    ],
    out_specs=[
        pl.BlockSpec((Bn, D), lambda i: (i, 0)),
        pl.BlockSpec((4, Bn, H), lambda i: (0, i, 0)),
    ],
    out_shape=[
        jax.ShapeDtypeStruct((N, D), jnp.float32),
        jax.ShapeDtypeStruct((4, N, H), jnp.float32),
    ],
)

_k2 = pl.pallas_call(
    _k2_body,
    grid=(NBLK,),
    in_specs=[
        pl.BlockSpec((Bn, D), lambda i: (i, 0)),
        _full((D, D)), _full((1, D)),
        *_agg_specs(), *_cnt_specs(),
        _full((2, D, D)),
    ],
    out_specs=[
        pl.BlockSpec((Bn, D), lambda i: (i, 0)),
        pl.BlockSpec((4, Bn, H), lambda i: (0, i, 0)),
    ],
    out_shape=[
        jax.ShapeDtypeStruct((N, D), jnp.float32),
        jax.ShapeDtypeStruct((4, N, H), jnp.float32),
    ],
)

_k3 = pl.pallas_call(
    _k3_body,
    grid=(NBLK,),
    in_specs=[
        pl.BlockSpec((Bn, D), lambda i: (i, 0)),
        _full((D, D)), _full((1, D)),
        *_agg_specs(), *_cnt_specs(),
        _full((D, D)), _full((1, D)), _full((D, 2)), _full((1, 2)),
    ],
    out_specs=pl.BlockSpec((Bn, 2), lambda i: (i, 0)),
    out_shape=jax.ShapeDtypeStruct((N, 2), jnp.float32),
)


def _sc_body(with_cnt, tab, gx, sx, *rest):
    if with_cnt:
        (agg_out, cnt_out, acc, cntacc, ones, gbuf, sbuf, rows,
         sem0, sem1, sem2) = rest
    else:
        (agg_out, acc, gbuf, sbuf, rows, sem0, sem1, sem2) = rest
        cnt_out = cntacc = ones = None
    sems = [sem0, sem1, sem2]
    c = lax.axis_index("c")
    s = lax.axis_index("s")

    def _zr(i, carry):
        def _zc(j, carry2):
            rows[0, i, pl.ds(j * 16, 16)] = jnp.zeros((16,), jnp.float32)
            return carry2
        return lax.fori_loop(0, H // 16, _zc, carry)
    lax.fori_loop(0, CH, _zr, 0)

    if with_cnt:
        def _zo(i, carry):
            ones[i, pl.ds(0, 16)] = jnp.zeros((16,), jnp.float32)
            return carry
        lax.fori_loop(0, CH, _zo, 0)

    def _za(j, carry):
        k = s + NS * j

        @pl.when(k < NZCH)
        def _():
            pltpu.sync_copy(rows.at[0], acc.at[pl.ds(k * CH, CH)])
            if with_cnt:
                pltpu.sync_copy(ones, cntacc.at[pl.ds(k * CH, CH)])
        return carry
    lax.fori_loop(0, (NZCH + NS - 1) // NS, _za, 0)

    if with_cnt:
        def _so(i, carry):
            ones[i, pl.ds(0, 16)] = jnp.ones((16,), jnp.float32)
            return carry
        lax.fori_loop(0, CH, _so, 0)

    plsc.subcore_barrier()

    def _outer(i0, carry):
        base = s * CPS + i0 * GRP
        pltpu.sync_copy(gx.at[c, pl.ds(base, GRP)], gbuf)
        pltpu.sync_copy(sx.at[pl.ds(base, GRP)], sbuf)
        descs = [None] * GRP
        for k in range(2):
            descs[k] = pltpu.async_copy(tab.at[gbuf.at[k]], rows.at[k % NB],
                                        sems[k % NB])
        for k in range(GRP):
            if k + 2 < GRP:
                descs[k + 2] = pltpu.async_copy(
                    tab.at[gbuf.at[k + 2]], rows.at[(k + 2) % NB],
                    sems[(k + 2) % NB])
            descs[k].wait()
            pltpu.sync_copy(rows.at[k % NB], acc.at[sbuf.at[k]], add=True)
            if with_cnt:
                pltpu.sync_copy(ones, cntacc.at[sbuf.at[k]], add=True)
        return carry
    lax.fori_loop(0, NGRP, _outer, 0)

    plsc.subcore_barrier()

    def _wb(j, carry):
        k = s + NS * j

        @pl.when(k < NWCH)
        def _():
            pltpu.sync_copy(acc.at[pl.ds(k * WCH, WCH)],
                            agg_out.at[c, pl.ds(k * WCH, WCH)])
            if with_cnt:
                pltpu.sync_copy(cntacc.at[pl.ds(k * WCH, WCH)],
                                cnt_out.at[c, pl.ds(k * WCH, WCH)])
        return carry
    lax.fori_loop(0, (NWCH + NS - 1) // NS, _wb, 0)


def _make_sc(with_cnt):
    out_type = [jax.ShapeDtypeStruct((2, 2 * N, H), jnp.float32)]
    scratch = [
        pltpu.VMEM_SHARED((ACC_R, H), jnp.float32),
    ]
    if with_cnt:
        out_type.append(jax.ShapeDtypeStruct((2, 2 * N, 16), jnp.float32))
        scratch += [
            pltpu.VMEM_SHARED((ACC_R, 16), jnp.float32),
            pltpu.VMEM((CH, 16), jnp.float32),
        ]
    scratch += [
        pltpu.VMEM((GRP, CH), jnp.int32),
        pltpu.VMEM((GRP, CH), jnp.int32),
        pltpu.VMEM((NB, CH, H), jnp.float32),
        pltpu.SemaphoreType.DMA,
        pltpu.SemaphoreType.DMA,
        pltpu.SemaphoreType.DMA,
    ]
    return pl.kernel(
        functools.partial(_sc_body, with_cnt),
        out_type=tuple(out_type) if with_cnt else out_type[0],
        mesh=plsc.VectorSubcoreMesh(core_axis_name="c", subcore_axis_name="s"),
        scratch_types=scratch,
        compiler_params=pltpu.CompilerParams(use_tc_tiling_on_sc=False),
    )


_sc1 = _make_sc(True)
_sc2 = _make_sc(False)


def kernel(des, tweet, num_prop, cat_prop, edge_index, edge_type,
           W_cat, b_cat, W_in, b_in, W_rel, W_root, b_rgcn,
           W_o1, b_o1, W_o2, b_o2):
    src = edge_index[0].astype(jnp.int32)
    dst = edge_index[1].astype(jnp.int32)
    et = edge_type.astype(jnp.int32)
    g0 = jnp.pad(et * N + src, (0, EPAD - E))
    gx = jnp.stack([g0, g0 + 2 * N]).reshape(2, NCH, CH)
    sx = jnp.pad(et * N + dst, (0, EPAD - E),
                 constant_values=2 * N).reshape(NCH, CH)

    bc = b_cat.reshape(1, D)
    bi = b_in.reshape(1, D)
    br = b_rgcn.reshape(1, D)
    bo1 = b_o1.reshape(1, D)
    bo2 = b_o2.reshape(1, 2)

    x1, t1 = _k1(cat_prop, W_cat, bc, W_in, bi, W_rel)
    agg1, cnt16 = _sc1(t1.reshape(4 * N, H), gx, sx)
    x2, t2 = _k2(x1, W_root, br, agg1, agg1, agg1, agg1, cnt16, cnt16, W_rel)
    agg2 = _sc2(t2.reshape(4 * N, H), gx, sx)
    return _k3(x2, W_root, br, agg2, agg2, agg2, agg2, cnt16, cnt16,
               W_o1, bo1, W_o2, bo2)


# speedup vs baseline: 11.1393x; 1.4123x over previous
import functools

import jax
import jax.numpy as jnp
from jax import lax
from jax.experimental import pallas as pl
from jax.experimental.pallas import tpu as pltpu
from jax.experimental.pallas import tpu_sc as plsc

N = 10000
E = 320000
D = 128
H = 64
Bn = 1000
NBLK = N // Bn

CH = 112
NS = 16
CPS = 180
NCH = CPS * NS
EPAD = NCH * CH
GRP = 12
NGRP = CPS // GRP
NB = 3
ACC_R = 2 * N + 8
NZF = 178
ZT = ACC_R - NZF * CH
WCH = 1000
NWCH = (2 * N) // WCH


def _leaky(v):
    return jnp.where(v >= 0, v, 0.01 * v)


def _table_write(x, wr_ref, t_ref):
    xr0 = jnp.dot(x, wr_ref[0])
    xr1 = jnp.dot(x, wr_ref[1])
    t_ref[0] = xr0[:, :H]
    t_ref[1] = xr1[:, :H]
    t_ref[2] = xr0[:, H:]
    t_ref[3] = xr1[:, H:]


def _k1_body(cat_ref, wc_ref, bc_ref, wi_ref, bi_ref, wr_ref, x_ref, t_ref):
    c = _leaky(jnp.dot(cat_ref[...], wc_ref[...]) + bc_ref[...])
    x = _leaky(jnp.dot(c, wi_ref[...]) + bi_ref[...])
    x_ref[...] = x
    _table_write(x, wr_ref, t_ref)


def _combine(x, wroot_ref, br_ref, a00, a01, a10, a11, c0, c1):
    cnt0 = jnp.maximum(c0[0][:, 0:1], 1.0)
    cnt1 = jnp.maximum(c1[0][:, 0:1], 1.0)
    lo = a00[0] / cnt0 + a01[0] / cnt1
    hi = a10[0] / cnt0 + a11[0] / cnt1
    return (jnp.dot(x, wroot_ref[...]) + br_ref[...]
            + jnp.concatenate([lo, hi], axis=1))


def _k2_body(x_ref, wroot_ref, br_ref, a00, a01, a10, a11, c0, c1, wr_ref,
             x2_ref, t_ref):
    x2 = _combine(x_ref[...], wroot_ref, br_ref, a00, a01, a10, a11, c0, c1)
    x2_ref[...] = x2
    _table_write(x2, wr_ref, t_ref)


def _k3_body(x_ref, wroot_ref, br_ref, a00, a01, a10, a11, c0, c1,
             wo1_ref, bo1_ref, wo2_ref, bo2_ref, out_ref):
    x3 = _combine(x_ref[...], wroot_ref, br_ref, a00, a01, a10, a11, c0, c1)
    x4 = _leaky(jnp.dot(x3, wo1_ref[...]) + bo1_ref[...])
    out_ref[...] = jnp.dot(x4, wo2_ref[...]) + bo2_ref[...]


def _full(shape):
    return pl.BlockSpec(shape, lambda i: (0,) * len(shape))


def _agg_specs():
    return [
        pl.BlockSpec((1, Bn, H), lambda i: (0, i, 0)),
        pl.BlockSpec((1, Bn, H), lambda i: (0, NBLK + i, 0)),
        pl.BlockSpec((1, Bn, H), lambda i: (1, i, 0)),
        pl.BlockSpec((1, Bn, H), lambda i: (1, NBLK + i, 0)),
    ]


def _cnt_specs():
    return [
        pl.BlockSpec((1, Bn, 16), lambda i: (0, i, 0)),
        pl.BlockSpec((1, Bn, 16), lambda i: (0, NBLK + i, 0)),
    ]


_k1 = pl.pallas_call(
    _k1_body,
    grid=(NBLK,),
    in_specs=[
        pl.BlockSpec((Bn, 11), lambda i: (i, 0)),
        _full((11, D)), _full((1, D)), _full((D, D)), _full((1, D)),
        _full((2, D, D)),
    ],
    out_specs=[
        pl.BlockSpec((Bn, D), lambda i: (i, 0)),
        pl.BlockSpec((4, Bn, H), lambda i: (0, i, 0)),
    ],
    out_shape=[
        jax.ShapeDtypeStruct((N, D), jnp.float32),
        jax.ShapeDtypeStruct((4, N, H), jnp.float32),
    ],
)

_k2 = pl.pallas_call(
    _k2_body,
    grid=(NBLK,),
    in_specs=[
        pl.BlockSpec((Bn, D), lambda i: (i, 0)),
        _full((D, D)), _full((1, D)),
        *_agg_specs(), *_cnt_specs(),
        _full((2, D, D)),
    ],
    out_specs=[
        pl.BlockSpec((Bn, D), lambda i: (i, 0)),
        pl.BlockSpec((4, Bn, H), lambda i: (0, i, 0)),
    ],
    out_shape=[
        jax.ShapeDtypeStruct((N, D), jnp.float32),
        jax.ShapeDtypeStruct((4, N, H), jnp.float32),
    ],
)

_k3 = pl.pallas_call(
    _k3_body,
    grid=(NBLK,),
    in_specs=[
        pl.BlockSpec((Bn, D), lambda i: (i, 0)),
        _full((D, D)), _full((1, D)),
        *_agg_specs(), *_cnt_specs(),
        _full((D, D)), _full((1, D)), _full((D, 2)), _full((1, 2)),
    ],
    out_specs=pl.BlockSpec((Bn, 2), lambda i: (i, 0)),
    out_shape=jax.ShapeDtypeStruct((N, 2), jnp.float32),
)


def _sc_body(with_cnt, tab, gx, sx, *rest):
    if with_cnt:
        (agg_out, cnt_out, acc, cntacc, ones, zb16, gbuf, sbuf, rows,
         g0, g1, g2, s0, s1, s2, cn) = rest
    else:
        (agg_out, acc, gbuf, sbuf, rows, g0, g1, g2, s0, s1, s2) = rest
        cnt_out = cntacc = ones = zb16 = cn = None
    gsem = [g0, g1, g2]
    ssem = [s0, s1, s2]
    c = lax.axis_index("c")
    s = lax.axis_index("s")

    def _zr(i, carry):
        def _zc(j, carry2):
            for b in range(NB):
                rows[b, i, pl.ds(j * 16, 16)] = jnp.zeros((16,), jnp.float32)
            return carry2
        return lax.fori_loop(0, H // 16, _zc, carry)
    lax.fori_loop(0, CH, _zr, 0)

    def _zs(i, carry):
        sbuf[0, pl.ds(i * 16, 16)] = jnp.zeros((16,), jnp.int32)
        return carry
    lax.fori_loop(0, CH // 16, _zs, 0)

    if with_cnt:
        def _zo(i, carry):
            zb16[i, pl.ds(0, 16)] = jnp.zeros((16,), jnp.float32)
            ones[i, pl.ds(0, 16)] = jnp.ones((16,), jnp.float32)
            return carry
        lax.fori_loop(0, CH, _zo, 0)

    def _za(j, carry):
        k = s + NS * j

        @pl.when(k < NZF)
        def _():
            pltpu.sync_copy(rows.at[0], acc.at[pl.ds(k * CH, CH)])
            if with_cnt:
                pltpu.sync_copy(zb16, cntacc.at[pl.ds(k * CH, CH)])

        @pl.when(k == NZF)
        def _():
            pltpu.sync_copy(rows.at[0, pl.ds(0, ZT)],
                            acc.at[pl.ds(NZF * CH, ZT)])
            if with_cnt:
                pltpu.sync_copy(zb16.at[pl.ds(0, ZT)],
                                cntacc.at[pl.ds(NZF * CH, ZT)])
        return carry
    lax.fori_loop(0, (NZF + NS) // NS + 1, _za, 0)

    plsc.subcore_barrier()

    for b in range(NB):
        pltpu.async_copy(rows.at[b], acc.at[sbuf.at[0]], ssem[b], add=True)

    def _outer(it, carry):
        base = s * CPS + it * GRP
        pltpu.sync_copy(gx.at[c, pl.ds(base, GRP)], gbuf)
        pltpu.sync_copy(sx.at[pl.ds(base, GRP)], sbuf)
        gd = [None] * GRP
        for k in range(2):
            pltpu.make_async_copy(rows.at[k], acc.at[sbuf.at[0]],
                                  ssem[k]).wait()
            gd[k] = pltpu.async_copy(tab.at[gbuf.at[k]], rows.at[k],
                                     gsem[k])
        for k in range(GRP):
            b = k % NB
            if k + 2 < GRP:
                b2 = (k + 2) % NB
                pltpu.make_async_copy(rows.at[b2], acc.at[sbuf.at[0]],
                                      ssem[b2]).wait()
                gd[k + 2] = pltpu.async_copy(tab.at[gbuf.at[k + 2]],
                                             rows.at[b2], gsem[b2])
            gd[k].wait()
            pltpu.async_copy(rows.at[b], acc.at[sbuf.at[k]], ssem[b],
                             add=True)
            if with_cnt:
                pltpu.async_copy(ones, cntacc.at[sbuf.at[k]], cn, add=True)
        return carry
    lax.fori_loop(0, NGRP, _outer, 0)

    for b in range(NB):
        pltpu.make_async_copy(rows.at[b], acc.at[sbuf.at[0]], ssem[b]).wait()
    if with_cnt:
        def _dr(i, carry):
            pltpu.make_async_copy(ones, cntacc.at[sbuf.at[0]], cn).wait()
            return carry
        lax.fori_loop(0, CPS, _dr, 0)

    plsc.subcore_barrier()

    def _wb(j, carry):
        k = s + NS * j

        @pl.when(k < NWCH)
        def _():
            pltpu.sync_copy(acc.at[pl.ds(k * WCH, WCH)],
                            agg_out.at[c, pl.ds(k * WCH, WCH)])
            if with_cnt:
                pltpu.sync_copy(cntacc.at[pl.ds(k * WCH, WCH)],
                                cnt_out.at[c, pl.ds(k * WCH, WCH)])
        return carry
    lax.fori_loop(0, (NWCH + NS - 1) // NS, _wb, 0)


def _make_sc(with_cnt):
    out_type = [jax.ShapeDtypeStruct((2, 2 * N, H), jnp.float32)]
    scratch = [
        pltpu.VMEM_SHARED((ACC_R, H), jnp.float32),
    ]
    if with_cnt:
        out_type.append(jax.ShapeDtypeStruct((2, 2 * N, 16), jnp.float32))
        scratch += [
            pltpu.VMEM_SHARED((ACC_R, 16), jnp.float32),
            pltpu.VMEM((CH, 16), jnp.float32),
            pltpu.VMEM((CH, 16), jnp.float32),
        ]
    scratch += [
        pltpu.VMEM((GRP, CH), jnp.int32),
        pltpu.VMEM((GRP, CH), jnp.int32),
        pltpu.VMEM((NB, CH, H), jnp.float32),
        pltpu.SemaphoreType.DMA,
        pltpu.SemaphoreType.DMA,
        pltpu.SemaphoreType.DMA,
        pltpu.SemaphoreType.DMA,
        pltpu.SemaphoreType.DMA,
        pltpu.SemaphoreType.DMA,
    ]
    if with_cnt:
        scratch.append(pltpu.SemaphoreType.DMA)
    return pl.kernel(
        functools.partial(_sc_body, with_cnt),
        out_type=tuple(out_type) if with_cnt else out_type[0],
        mesh=plsc.VectorSubcoreMesh(core_axis_name="c", subcore_axis_name="s"),
        scratch_types=scratch,
        compiler_params=pltpu.CompilerParams(use_tc_tiling_on_sc=False),
    )


_sc1 = _make_sc(True)
_sc2 = _make_sc(False)


def kernel(des, tweet, num_prop, cat_prop, edge_index, edge_type,
           W_cat, b_cat, W_in, b_in, W_rel, W_root, b_rgcn,
           W_o1, b_o1, W_o2, b_o2):
    src = edge_index[0].astype(jnp.int32)
    dst = edge_index[1].astype(jnp.int32)
    et = edge_type.astype(jnp.int32)
    g0 = jnp.pad(et * N + src, (0, EPAD - E))
    gx = jnp.stack([g0, g0 + 2 * N]).reshape(2, NCH, CH)
    sx = jnp.pad(et * N + dst, (0, EPAD - E),
                 constant_values=2 * N).reshape(NCH, CH)

    bc = b_cat.reshape(1, D)
    bi = b_in.reshape(1, D)
    br = b_rgcn.reshape(1, D)
    bo1 = b_o1.reshape(1, D)
    bo2 = b_o2.reshape(1, 2)

    x1, t1 = _k1(cat_prop, W_cat, bc, W_in, bi, W_rel)
    agg1, cnt16 = _sc1(t1.reshape(4 * N, H), gx, sx)
    x2, t2 = _k2(x1, W_root, br, agg1, agg1, agg1, agg1, cnt16, cnt16, W_rel)
    agg2 = _sc2(t2.reshape(4 * N, H), gx, sx)
    return _k3(x2, W_root, br, agg2, agg2, agg2, agg2, cnt16, cnt16,
               W_o1, bo1, W_o2, bo2)


# speedup vs baseline: 11.1669x; 1.0025x over previous
import functools

import jax
import jax.numpy as jnp
from jax import lax
from jax.experimental import pallas as pl
from jax.experimental.pallas import tpu as pltpu
from jax.experimental.pallas import tpu_sc as plsc

N = 10000
E = 320000
D = 128
H = 64
Bn = 1000
NBLK = N // Bn

CH = 112
NS = 16
CPS = 180
NCH = CPS * NS
EPAD = NCH * CH
GRP = 12
NGRP = CPS // GRP
NB = 3
ACC_R = 2 * N + 8
NZF = 178
ZT = ACC_R - NZF * CH
WCH = 1000
NWCH = (2 * N) // WCH


def _leaky(v):
    return jnp.where(v >= 0, v, 0.01 * v)


def _table_write(x, wr_ref, t_ref):
    xr0 = jnp.dot(x, wr_ref[0])
    xr1 = jnp.dot(x, wr_ref[1])
    t_ref[0] = xr0[:, :H]
    t_ref[1] = xr1[:, :H]
    t_ref[2] = xr0[:, H:]
    t_ref[3] = xr1[:, H:]


def _k1_body(cat_ref, wc_ref, bc_ref, wi_ref, bi_ref, wr_ref, x_ref, t_ref):
    c = _leaky(jnp.dot(cat_ref[...], wc_ref[...]) + bc_ref[...])
    x = _leaky(jnp.dot(c, wi_ref[...]) + bi_ref[...])
    x_ref[...] = x
    _table_write(x, wr_ref, t_ref)


def _combine(x, wroot_ref, br_ref, a00, a01, a10, a11, c0, c1):
    cnt0 = jnp.maximum(c0[0][:, 0:1], 1.0)
    cnt1 = jnp.maximum(c1[0][:, 0:1], 1.0)
    lo = a00[0] / cnt0 + a01[0] / cnt1
    hi = a10[0] / cnt0 + a11[0] / cnt1
    return (jnp.dot(x, wroot_ref[...]) + br_ref[...]
            + jnp.concatenate([lo, hi], axis=1))


def _k2_body(x_ref, wroot_ref, br_ref, a00, a01, a10, a11, c0, c1, wr_ref,
             x2_ref, t_ref):
    x2 = _combine(x_ref[...], wroot_ref, br_ref, a00, a01, a10, a11, c0, c1)
    x2_ref[...] = x2
    _table_write(x2, wr_ref, t_ref)


def _k3_body(x_ref, wroot_ref, br_ref, a00, a01, a10, a11, c0, c1,
             wo1_ref, bo1_ref, wo2_ref, bo2_ref, out_ref):
    x3 = _combine(x_ref[...], wroot_ref, br_ref, a00, a01, a10, a11, c0, c1)
    x4 = _leaky(jnp.dot(x3, wo1_ref[...]) + bo1_ref[...])
    out_ref[...] = jnp.dot(x4, wo2_ref[...]) + bo2_ref[...]


def _full(shape):
    return pl.BlockSpec(shape, lambda i: (0,) * len(shape))


def _agg_specs():
    return [
        pl.BlockSpec((1, Bn, H), lambda i: (0, i, 0)),
        pl.BlockSpec((1, Bn, H), lambda i: (0, NBLK + i, 0)),
        pl.BlockSpec((1, Bn, H), lambda i: (1, i, 0)),
        pl.BlockSpec((1, Bn, H), lambda i: (1, NBLK + i, 0)),
    ]


def _cnt_specs():
    return [
        pl.BlockSpec((1, Bn, 16), lambda i: (0, i, 0)),
        pl.BlockSpec((1, Bn, 16), lambda i: (0, NBLK + i, 0)),
    ]


_k1 = pl.pallas_call(
    _k1_body,
    grid=(NBLK,),
    in_specs=[
        pl.BlockSpec((Bn, 11), lambda i: (i, 0)),
        _full((11, D)), _full((1, D)), _full((D, D)), _full((1, D)),
        _full((2, D, D)),
    ],
    out_specs=[
        pl.BlockSpec((Bn, D), lambda i: (i, 0)),
        pl.BlockSpec((4, Bn, H), lambda i: (0, i, 0)),
    ],
    out_shape=[
        jax.ShapeDtypeStruct((N, D), jnp.float32),
        jax.ShapeDtypeStruct((4, N, H), jnp.float32),
    ],
)

_k2 = pl.pallas_call(
    _k2_body,
    grid=(NBLK,),
    in_specs=[
        pl.BlockSpec((Bn, D), lambda i: (i, 0)),
        _full((D, D)), _full((1, D)),
        *_agg_specs(), *_cnt_specs(),
        _full((2, D, D)),
    ],
    out_specs=[
        pl.BlockSpec((Bn, D), lambda i: (i, 0)),
        pl.BlockSpec((4, Bn, H), lambda i: (0, i, 0)),
    ],
    out_shape=[
        jax.ShapeDtypeStruct((N, D), jnp.float32),
        jax.ShapeDtypeStruct((4, N, H), jnp.float32),
    ],
)

_k3 = pl.pallas_call(
    _k3_body,
    grid=(NBLK,),
    in_specs=[
        pl.BlockSpec((Bn, D), lambda i: (i, 0)),
        _full((D, D)), _full((1, D)),
        *_agg_specs(), *_cnt_specs(),
        _full((D, D)), _full((1, D)), _full((D, 2)), _full((1, 2)),
    ],
    out_specs=pl.BlockSpec((Bn, 2), lambda i: (i, 0)),
    out_shape=jax.ShapeDtypeStruct((N, 2), jnp.float32),
)


def _sc_body(with_cnt, nb, a, tab, gx, sx, *rest):
    if with_cnt:
        (agg_out, cnt_out, acc, cntacc, ones, zb16, gbuf, sbuf,
         rows) = rest[:9]
        gsem = list(rest[9:9 + nb])
        ssem = list(rest[9 + nb:9 + 2 * nb])
        cn = rest[9 + 2 * nb]
    else:
        (agg_out, acc, gbuf, sbuf, rows) = rest[:5]
        gsem = list(rest[5:5 + nb])
        ssem = list(rest[5 + nb:5 + 2 * nb])
        cnt_out = cntacc = ones = zb16 = cn = None
    c = lax.axis_index("c")
    s = lax.axis_index("s")

    def _zr(i, carry):
        def _zc(j, carry2):
            for b in range(nb):
                rows[b, i, pl.ds(j * 16, 16)] = jnp.zeros((16,), jnp.float32)
            return carry2
        return lax.fori_loop(0, H // 16, _zc, carry)
    lax.fori_loop(0, CH, _zr, 0)

    def _zs(i, carry):
        sbuf[0, pl.ds(i * 16, 16)] = jnp.zeros((16,), jnp.int32)
        return carry
    lax.fori_loop(0, CH // 16, _zs, 0)

    if with_cnt:
        def _zo(i, carry):
            zb16[i, pl.ds(0, 16)] = jnp.zeros((16,), jnp.float32)
            ones[i, pl.ds(0, 16)] = jnp.ones((16,), jnp.float32)
            return carry
        lax.fori_loop(0, CH, _zo, 0)

    def _za(j, carry):
        k = s + NS * j

        @pl.when(k < NZF)
        def _():
            pltpu.sync_copy(rows.at[0], acc.at[pl.ds(k * CH, CH)])
            if with_cnt:
                pltpu.sync_copy(zb16, cntacc.at[pl.ds(k * CH, CH)])

        @pl.when(k == NZF)
        def _():
            pltpu.sync_copy(rows.at[0, pl.ds(0, ZT)],
                            acc.at[pl.ds(NZF * CH, ZT)])
            if with_cnt:
                pltpu.sync_copy(zb16.at[pl.ds(0, ZT)],
                                cntacc.at[pl.ds(NZF * CH, ZT)])
        return carry
    lax.fori_loop(0, (NZF + NS) // NS + 1, _za, 0)

    plsc.subcore_barrier()

    for b in range(nb):
        pltpu.async_copy(rows.at[b], acc.at[sbuf.at[0]], ssem[b], add=True)

    def _outer(it, carry):
        base = s * CPS + it * GRP
        pltpu.sync_copy(gx.at[c, pl.ds(base, GRP)], gbuf)
        pltpu.sync_copy(sx.at[pl.ds(base, GRP)], sbuf)
        gd = [None] * GRP
        for k in range(a):
            b = k % nb
            pltpu.make_async_copy(rows.at[b], acc.at[sbuf.at[0]],
                                  ssem[b]).wait()
            gd[k] = pltpu.async_copy(tab.at[gbuf.at[k]], rows.at[b],
                                     gsem[b])
        for k in range(GRP):
            b = k % nb
            if k + a < GRP:
                b2 = (k + a) % nb
                pltpu.make_async_copy(rows.at[b2], acc.at[sbuf.at[0]],
                                      ssem[b2]).wait()
                gd[k + a] = pltpu.async_copy(tab.at[gbuf.at[k + a]],
                                             rows.at[b2], gsem[b2])
            gd[k].wait()
            pltpu.async_copy(rows.at[b], acc.at[sbuf.at[k]], ssem[b],
                             add=True)
            if with_cnt:
                pltpu.async_copy(ones, cntacc.at[sbuf.at[k]], cn, add=True)
        return carry
    lax.fori_loop(0, NGRP, _outer, 0)

    for b in range(nb):
        pltpu.make_async_copy(rows.at[b], acc.at[sbuf.at[0]], ssem[b]).wait()
    if with_cnt:
        def _dr(i, carry):
            pltpu.make_async_copy(ones, cntacc.at[sbuf.at[0]], cn).wait()
            return carry
        lax.fori_loop(0, CPS, _dr, 0)

    plsc.subcore_barrier()

    def _wb(j, carry):
        k = s + NS * j

        @pl.when(k < NWCH)
        def _():
            pltpu.sync_copy(acc.at[pl.ds(k * WCH, WCH)],
                            agg_out.at[c, pl.ds(k * WCH, WCH)])
            if with_cnt:
                pltpu.sync_copy(cntacc.at[pl.ds(k * WCH, WCH)],
                                cnt_out.at[c, pl.ds(k * WCH, WCH)])
        return carry
    lax.fori_loop(0, (NWCH + NS - 1) // NS, _wb, 0)


def _make_sc(with_cnt, nb, a):
    out_type = [jax.ShapeDtypeStruct((2, 2 * N, H), jnp.float32)]
    scratch = [
        pltpu.VMEM_SHARED((ACC_R, H), jnp.float32),
    ]
    if with_cnt:
        out_type.append(jax.ShapeDtypeStruct((2, 2 * N, 16), jnp.float32))
        scratch += [
            pltpu.VMEM_SHARED((ACC_R, 16), jnp.float32),
            pltpu.VMEM((CH, 16), jnp.float32),
            pltpu.VMEM((CH, 16), jnp.float32),
        ]
    scratch += [
        pltpu.VMEM((GRP, CH), jnp.int32),
        pltpu.VMEM((GRP, CH), jnp.int32),
        pltpu.VMEM((nb, CH, H), jnp.float32),
    ]
    scratch += [pltpu.SemaphoreType.DMA] * (2 * nb)
    if with_cnt:
        scratch.append(pltpu.SemaphoreType.DMA)
    return pl.kernel(
        functools.partial(_sc_body, with_cnt, nb, a),
        out_type=tuple(out_type) if with_cnt else out_type[0],
        mesh=plsc.VectorSubcoreMesh(core_axis_name="c", subcore_axis_name="s"),
        scratch_types=scratch,
        compiler_params=pltpu.CompilerParams(use_tc_tiling_on_sc=False),
    )


_sc1 = _make_sc(True, 3, 2)
_sc2 = _make_sc(False, 6, 4)


def kernel(des, tweet, num_prop, cat_prop, edge_index, edge_type,
           W_cat, b_cat, W_in, b_in, W_rel, W_root, b_rgcn,
           W_o1, b_o1, W_o2, b_o2):
    src = edge_index[0].astype(jnp.int32)
    dst = edge_index[1].astype(jnp.int32)
    et = edge_type.astype(jnp.int32)
    g0 = jnp.pad(et * N + src, (0, EPAD - E))
    gx = jnp.stack([g0, g0 + 2 * N]).reshape(2, NCH, CH)
    sx = jnp.pad(et * N + dst, (0, EPAD - E),
                 constant_values=2 * N).reshape(NCH, CH)

    bc = b_cat.reshape(1, D)
    bi = b_in.reshape(1, D)
    br = b_rgcn.reshape(1, D)
    bo1 = b_o1.reshape(1, D)
    bo2 = b_o2.reshape(1, 2)

    x1, t1 = _k1(cat_prop, W_cat, bc, W_in, bi, W_rel)
    agg1, cnt16 = _sc1(t1.reshape(4 * N, H), gx, sx)
    x2, t2 = _k2(x1, W_root, br, agg1, agg1, agg1, agg1, cnt16, cnt16, W_rel)
    agg2 = _sc2(t2.reshape(4 * N, H), gx, sx)
    return _k3(x2, W_root, br, agg2, agg2, agg2, agg2, cnt16, cnt16,
               W_o1, bo1, W_o2, bo2)
